# Initial kernel scaffold; baseline (speedup 1.0000x reference)
import jax
import jax.numpy as jnp
from jax.experimental import pallas as pl


def kernel(xyz, params):
    raise NotImplementedError("write your pallas kernel here")


# speedup vs baseline: 16.1915x; 16.1915x over previous
import functools

import jax
import jax.numpy as jnp
import numpy as np
from jax import lax
from jax.experimental import pallas as pl
from jax.experimental.pallas import tpu as pltpu
from jax.experimental.pallas import tpu_sc as plsc

EPS_BN = 1e-5
F32 = jnp.float32
I32 = jnp.int32
BIG = 1e9


SQ = np.float32(np.sqrt(np.float32(1.0 + EPS_BN)))


def _prep(layers):
    out = []
    for W, b, g, bt in layers:
        bgb = jnp.concatenate([b.reshape(1, -1), g.reshape(1, -1),
                               bt.reshape(1, -1)], axis=0)
        out.append((W.T, bgb))
    return out


def _bnrelu(h, bgb):
    return jnp.maximum(h / SQ * bgb[1:2] + bgb[2:3], 0.0)


def _dot(a, b, prec=None):
    return lax.dot_general(a, b, (((1,), (0,)), ((), ())),
                           preferred_element_type=F32, precision=prec)


def _fps_body(npoint, x_ref, y_ref, z_ref, cx_ref, cy_ref, cz_ref):
    x = x_ref[...]
    y = y_ref[...]
    z = z_ref[...]
    B, N = x.shape
    iota = lax.broadcasted_iota(I32, (B, N), 1)
    iota_o = lax.broadcasted_iota(I32, (B, npoint), 1)

    def step(t, carry):
        dist, far, ox, oy, oz = carry
        oh = iota == far
        cx = jnp.sum(jnp.where(oh, x, 0.0), axis=1, keepdims=True)
        cy = jnp.sum(jnp.where(oh, y, 0.0), axis=1, keepdims=True)
        cz = jnp.sum(jnp.where(oh, z, 0.0), axis=1, keepdims=True)
        ox = jnp.where(iota_o == t, cx, ox)
        oy = jnp.where(iota_o == t, cy, oy)
        oz = jnp.where(iota_o == t, cz, oz)
        dx = x - cx
        dy = y - cy
        dz = z - cz
        d = dx * dx + dy * dy + dz * dz
        dist = jnp.minimum(dist, d)
        m = jnp.max(dist, axis=1, keepdims=True)
        nxt = jnp.min(jnp.where(dist == m, iota, N), axis=1, keepdims=True)
        return dist, nxt, ox, oy, oz

    init = (jnp.full((B, N), 1e10, F32), jnp.zeros((B, 1), I32),
            jnp.zeros((B, npoint), F32), jnp.zeros((B, npoint), F32),
            jnp.zeros((B, npoint), F32))
    _, _, ox, oy, oz = lax.fori_loop(0, npoint, step, init)
    cx_ref[...] = ox
    cy_ref[...] = oy
    cz_ref[...] = oz


def _fps(x, y, z, npoint):
    B, N = x.shape
    shp = jax.ShapeDtypeStruct((B, npoint), F32)
    return pl.pallas_call(
        functools.partial(_fps_body, npoint),
        out_shape=(shp, shp, shp),
    )(x, y, z)


def _ballq_body(N, S, ns, r2, xr_ref, yr_ref, zr_ref, cxc_ref, cyc_ref,
                czc_ref, idx_ref):
    b = pl.program_id(0)
    xr = xr_ref[0]
    yr = yr_ref[0]
    zr = zr_ref[0]
    cxc = cxc_ref[0]
    cyc = cyc_ref[0]
    czc = czc_ref[0]

    zsc = jnp.zeros((S, 5), F32)
    c8 = jnp.concatenate([cxc, cyc, czc, zsc], axis=1)
    zsn = jnp.zeros((5, N), F32)
    x8 = jnp.concatenate([xr, yr, zr, zsn], axis=0)
    dot = _dot(c8, x8)
    c2 = cxc * cxc + cyc * cyc + czc * czc
    x2 = xr * xr + yr * yr + zr * zr
    d2 = c2 + x2 - 2.0 * dot

    giota = lax.broadcasted_iota(I32, (S, N), 1) + b * N
    key = jnp.where(d2 <= r2, giota.astype(F32), BIG)
    acc = jnp.zeros((S, ns), F32)
    lane = lax.broadcasted_iota(I32, (S, ns), 1)
    first = None
    for k in range(ns):
        mk = jnp.min(key, axis=1, keepdims=True)
        if first is None:
            first = mk
            col = mk
        else:
            col = jnp.where(mk < BIG, mk, first)
        acc = jnp.where(lane == k, col, acc)
        key = jnp.where(key == mk, BIG, key)
    idx_ref[...] = acc.astype(I32)[None]


def _ballq(x, y, z, cx, cy, cz, r, ns):
    B, N = x.shape
    S = cx.shape[1]
    xr = x.reshape(B, 1, N)
    yr = y.reshape(B, 1, N)
    zr = z.reshape(B, 1, N)
    cxc = cx.reshape(B, S, 1)
    cyc = cy.reshape(B, S, 1)
    czc = cz.reshape(B, S, 1)
    spec_1n = pl.BlockSpec((1, 1, N), lambda b: (b, 0, 0))
    spec_s1 = pl.BlockSpec((1, S, 1), lambda b: (b, 0, 0))
    return pl.pallas_call(
        functools.partial(_ballq_body, N, S, ns, r * r),
        grid=(B,),
        in_specs=[spec_1n, spec_1n, spec_1n, spec_s1, spec_s1, spec_s1],
        out_specs=pl.BlockSpec((1, S, ns), lambda b: (b, 0, 0)),
        out_shape=jax.ShapeDtypeStruct((B, S, ns), I32),
    )(xr, yr, zr, cxc, cyc, czc)


def _sc_gather(table, idx):
    T, D = table.shape
    (Btot,) = idx.shape
    info = plsc.get_sparse_core_info()
    NC, NS = info.num_cores, info.num_subcores
    NW = NC * NS
    per_w = Btot // NW
    CH = 128
    n_ch = per_w // CH
    mesh = plsc.VectorSubcoreMesh(core_axis_name="c", subcore_axis_name="s")

    @functools.partial(
        pl.kernel,
        out_type=jax.ShapeDtypeStruct((Btot, D), F32),
        mesh=mesh,
        scratch_types=[
            pltpu.VMEM((CH,), I32),
            pltpu.VMEM((CH, D), F32),
            pltpu.SemaphoreType.DMA,
        ],
    )
    def k(table_hbm, idx_hbm, out_hbm, idx_v, rows_v, sem):
        wid = lax.axis_index("s") * NC + lax.axis_index("c")
        base = wid * per_w

        def body(g, carry):
            off = base + g * CH
            pltpu.sync_copy(idx_hbm.at[pl.ds(off, CH)], idx_v)
            pltpu.async_copy(table_hbm.at[idx_v], rows_v, sem).wait()
            pltpu.sync_copy(rows_v, out_hbm.at[pl.ds(off, CH)])
            return carry

        lax.fori_loop(0, n_ch, body, 0)

    return k(table, idx)


def _gmlp_body(st, ns, nf, g_ref, cx_ref, cy_ref, cz_ref, w1_ref, bgb1_ref,
               w2_ref, bgb2_ref, w3_ref, bgb3_ref, o_ref):
    g = g_ref[...]
    R_ = st * ns
    cols = []
    for c_ in (cx_ref, cy_ref, cz_ref):
        cols.append(jnp.broadcast_to(c_[0].reshape(st, 1, 1),
                                     (st, ns, 1)).reshape(R_, 1))
    gx = g[:, 0:1] - cols[0]
    gy = g[:, 1:2] - cols[1]
    gz = g[:, 2:3] - cols[2]
    if nf == 3:
        feat = jnp.concatenate(
            [gx, gy, gz, g[:, 0:3], jnp.zeros((R_, 2), F32)], axis=1)
    else:
        feat = jnp.concatenate(
            [gx, gy, gz, g[:, 3:3 + nf], jnp.zeros((R_, 5), F32)], axis=1)
    h = _dot(feat, w1_ref[...]) + bgb1_ref[0:1]
    h = _bnrelu(h, bgb1_ref[...])
    h = _bnrelu(_dot(h, w2_ref[...]) + bgb2_ref[0:1], bgb2_ref[...])
    h = _bnrelu(_dot(h, w3_ref[...]) + bgb3_ref[0:1], bgb3_ref[...])
    C3 = h.shape[1]
    o_ref[...] = jnp.max(h.reshape(st, ns, C3), axis=1)[None]


def _gmlp(G, cx, cy, cz, nf, ns, st, w1, bgb1, w2, bgb2, w3, bgb3):
    B, S = cx.shape
    Dg = G.shape[1]
    C3 = w3.shape[1]
    nt = S // st

    def wspec(a):
        return pl.BlockSpec(a.shape, lambda b, t, nd=a.ndim: (0,) * nd)

    spec_c = pl.BlockSpec((1, st, 1), lambda b, t: (b, t, 0))
    return pl.pallas_call(
        functools.partial(_gmlp_body, st, ns, nf),
        grid=(B, nt),
        in_specs=[
            pl.BlockSpec((st * ns, Dg), lambda b, t: (b * (S // st) + t, 0)),
            spec_c, spec_c, spec_c,
            wspec(w1), wspec(bgb1), wspec(w2), wspec(bgb2), wspec(w3),
            wspec(bgb3),
        ],
        out_specs=pl.BlockSpec((1, st, C3), lambda b, t: (b, t, 0)),
        out_shape=jax.ShapeDtypeStruct((B, S, C3), F32),
    )(G, cx.reshape(B, S, 1), cy.reshape(B, S, 1), cz.reshape(B, S, 1),
      w1, bgb1, w2, bgb2, w3, bgb3)


def _sa3fp3_body(cxc_ref, cyc_ref, czc_ref, l2_ref, w1_ref, b1_ref,
                 w2_ref, b2_ref, w3_ref, b3_ref, fwa_ref, fwb_ref, fb1_ref,
                 fw2_ref, fb2_ref, o_ref):
    l2 = l2_ref[0]
    cxc = cxc_ref[0]
    cyc = cyc_ref[0]
    czc = czc_ref[0]
    feat = jnp.concatenate([cxc, cyc, czc, l2,
                            jnp.zeros((l2.shape[0], 5), F32)], axis=1)
    h = _dot(feat, w1_ref[...]) + b1_ref[0:1]
    h = _bnrelu(h, b1_ref[...])
    h = _bnrelu(_dot(h, w2_ref[...]) + b2_ref[0:1], b2_ref[...])
    h = _bnrelu(_dot(h, w3_ref[...]) + b3_ref[0:1], b3_ref[...])
    v = jnp.max(h, axis=0, keepdims=True)
    f = _dot(l2, fwa_ref[...]) + _dot(v, fwb_ref[...]) + fb1_ref[0:1]
    f = _bnrelu(f, fb1_ref[...])
    f = _bnrelu(_dot(f, fw2_ref[...]) + fb2_ref[0:1], fb2_ref[...])
    o_ref[...] = f[None]


def _sa3fp3(c2x, c2y, c2z, l2, w1, b1, w2, b2, w3, b3, fwa, fwb, fb1,
            fw2, fb2):
    B, S = c2x.shape
    cxc = c2x.reshape(B, S, 1)
    cyc = c2y.reshape(B, S, 1)
    czc = c2z.reshape(B, S, 1)
    spec_s1 = pl.BlockSpec((1, S, 1), lambda b: (b, 0, 0))

    def wspec(a):
        return pl.BlockSpec(a.shape, lambda b, nd=a.ndim: (0,) * nd)

    args = [cxc, cyc, czc, l2, w1, b1, w2, b2, w3, b3, fwa, fwb, fb1,
            fw2, fb2]
    in_specs = [spec_s1, spec_s1, spec_s1,
                pl.BlockSpec((1, S, 256), lambda b: (b, 0, 0))]
    in_specs += [wspec(a) for a in args[4:]]
    return pl.pallas_call(
        _sa3fp3_body, grid=(B,), in_specs=in_specs,
        out_specs=pl.BlockSpec((1, S, 256), lambda b: (b, 0, 0)),
        out_shape=jax.ShapeDtypeStruct((B, S, 256), F32),
    )(*args)


def _fp_body(Nq, Nr, nmm, head, qxc_ref, qyc_ref, qzc_ref, rxr_ref, ryr_ref,
             rzr_ref, pts_ref, skipw_refs, o_ref):
    pts = pts_ref[0]
    qxc = qxc_ref[0]
    qyc = qyc_ref[0]
    qzc = qzc_ref[0]
    rxr = rxr_ref[0]
    ryr = ryr_ref[0]
    rzr = rzr_ref[0]
    c8 = jnp.concatenate([qxc, qyc, qzc, jnp.zeros((Nq, 5), F32)], axis=1)
    x8 = jnp.concatenate([rxr, ryr, rzr, jnp.zeros((5, Nr), F32)], axis=0)
    dot = _dot(c8, x8)
    q2 = qxc * qxc + qyc * qyc + qzc * qzc
    r2 = rxr * rxr + ryr * ryr + rzr * rzr
    d2 = q2 + r2 - 2.0 * dot

    iota = lax.broadcasted_iota(I32, (Nq, Nr), 1)
    d = d2
    idxs, ds = [], []
    for _ in range(3):
        m = jnp.min(d, axis=1, keepdims=True)
        i = jnp.min(jnp.where(d == m, iota, Nr), axis=1, keepdims=True)
        idxs.append(i)
        ds.append(m)
        d = jnp.where(iota == i, jnp.inf, d)
    rec = [1.0 / (dk + 1e-8) for dk in ds]
    tot = rec[0] + rec[1] + rec[2]
    M = jnp.zeros((Nq, Nr), F32)
    for j in range(3):
        M = M + jnp.where(iota == idxs[j], rec[j] / tot, 0.0)
    interp = _dot(M, pts, lax.Precision.HIGHEST)
    return interp


def _fp2_body(Nq, Nr, qxc_ref, qyc_ref, qzc_ref, rxr_ref, ryr_ref, rzr_ref,
              pts_ref, skip_ref, wa_ref, wb_ref, b1_ref, w2_ref, b2_ref,
              o_ref):
    interp = _fp_body(Nq, Nr, None, None, qxc_ref, qyc_ref, qzc_ref, rxr_ref,
                      ryr_ref, rzr_ref, pts_ref, None, None)
    h = _dot(skip_ref[0], wa_ref[...]) + _dot(interp, wb_ref[...]) + b1_ref[0:1]
    h = _bnrelu(h, b1_ref[...])
    h = _bnrelu(_dot(h, w2_ref[...]) + b2_ref[0:1], b2_ref[...])
    o_ref[...] = h[None]


def _fp1_body(Nq, Nr, qxc_ref, qyc_ref, qzc_ref, rxr_ref, ryr_ref, rzr_ref,
              pts_ref, wxyz_ref, wc_ref, b1_ref, w2_ref, b2_ref, w3_ref,
              b3_ref, wc1_ref, bc1_ref, wc2_ref, bc2_ref, o_ref):
    interp = _fp_body(Nq, Nr, None, None, qxc_ref, qyc_ref, qzc_ref, rxr_ref,
                      ryr_ref, rzr_ref, pts_ref, None, None)
    wxyz = wxyz_ref[...]
    qxc = qxc_ref[0]
    qyc = qyc_ref[0]
    qzc = qzc_ref[0]
    h = _dot(interp, wc_ref[...]) + b1_ref[0:1]
    h = h + qxc * wxyz[0:1] + qyc * wxyz[1:2] + qzc * wxyz[2:3]
    h = _bnrelu(h, b1_ref[...])
    h = _bnrelu(_dot(h, w2_ref[...]) + b2_ref[0:1], b2_ref[...])
    h = _bnrelu(_dot(h, w3_ref[...]) + b3_ref[0:1], b3_ref[...])
    h = _bnrelu(_dot(h, wc1_ref[...]) + bc1_ref[0:1], bc1_ref[...])
    h = _dot(h, wc2_ref[...]) + bc2_ref[...]
    o_ref[...] = h[None]


def _fp_call(body, qx, qy, qz, rx, ry, rz, pts, extra, Cout):
    B, Nq = qx.shape
    Nr = rx.shape[1]
    Cp = pts.shape[-1]
    qxc = qx.reshape(B, Nq, 1)
    qyc = qy.reshape(B, Nq, 1)
    qzc = qz.reshape(B, Nq, 1)
    rxr = rx.reshape(B, 1, Nr)
    ryr = ry.reshape(B, 1, Nr)
    rzr = rz.reshape(B, 1, Nr)
    spec_q = pl.BlockSpec((1, Nq, 1), lambda b: (b, 0, 0))
    spec_r = pl.BlockSpec((1, 1, Nr), lambda b: (b, 0, 0))
    in_specs = [spec_q, spec_q, spec_q, spec_r, spec_r, spec_r,
                pl.BlockSpec((1, Nr, Cp), lambda b: (b, 0, 0))]
    args = [qxc, qyc, qzc, rxr, ryr, rzr, pts]
    for a in extra:
        if a.ndim == 3:
            in_specs.append(pl.BlockSpec((1,) + a.shape[1:],
                                         lambda b: (b, 0, 0)))
        else:
            in_specs.append(pl.BlockSpec(a.shape, lambda b, nd=a.ndim: (0,) * nd))
        args.append(a)
    return pl.pallas_call(
        functools.partial(body, Nq, Nr), grid=(B,), in_specs=in_specs,
        out_specs=pl.BlockSpec((1, Nq, Cout), lambda b: (b, 0, 0)),
        out_shape=jax.ShapeDtypeStruct((B, Nq, Cout), F32),
    )(*args)


def kernel(xyz, params):
    B, N, _ = xyz.shape
    x = xyz[..., 0]
    y = xyz[..., 1]
    z = xyz[..., 2]

    sa1 = _prep(params['sa1'])
    sa2 = _prep(params['sa2'])
    sa3 = _prep(params['sa3'])
    fp3 = _prep(params['fp3'])
    fp2 = _prep(params['fp2'])
    fp1 = _prep(params['fp1'])
    Wc1, bc1, gc1, btc1 = params['conv1']
    cbgb1 = jnp.concatenate([bc1.reshape(1, -1), gc1.reshape(1, -1),
                             btc1.reshape(1, -1)], axis=0)
    Wc2, bc2 = params['conv2']

    c1x, c1y, c1z = _fps(x, y, z, 512)
    W1t, bgb11 = sa1[0]
    w11 = jnp.concatenate([W1t, jnp.zeros((2, W1t.shape[1]), F32)], 0)
    idx1 = _ballq(x, y, z, c1x, c1y, c1z, 0.2, 32)
    T1 = jnp.pad(jnp.stack([x, y, z], -1).reshape(B * N, 3),
                 ((0, 0), (0, 125)))
    G1 = _sc_gather(T1, idx1.reshape(-1))
    l1 = _gmlp(G1, c1x, c1y, c1z, 3, 32, 128, w11, bgb11,
               sa1[1][0], sa1[1][1], sa1[2][0], sa1[2][1])

    c2x, c2y, c2z = _fps(c1x, c1y, c1z, 128)
    W2t, bgb21 = sa2[0]
    w21 = jnp.concatenate([W2t, jnp.zeros((5, W2t.shape[1]), F32)], 0)
    idx2 = _ballq(c1x, c1y, c1z, c2x, c2y, c2z, 0.4, 64)
    T2 = jnp.pad(jnp.concatenate(
        [jnp.stack([c1x, c1y, c1z], -1), l1], -1).reshape(B * 512, 131),
        ((0, 0), (0, 125)))
    G2 = _sc_gather(T2, idx2.reshape(-1))
    l2 = _gmlp(G2, c2x, c2y, c2z, 128, 64, 64, w21, bgb21,
               sa2[1][0], sa2[1][1], sa2[2][0], sa2[2][1])

    W31t, bgb31 = sa3[0]
    w31 = jnp.concatenate([W31t, jnp.zeros((5, W31t.shape[1]), F32)], 0)
    Wf1t, fbgb1 = fp3[0]
    l2p = _sa3fp3(c2x, c2y, c2z, l2,
                  w31, bgb31,
                  sa3[1][0], sa3[1][1],
                  sa3[2][0], sa3[2][1],
                  Wf1t[:256], Wf1t[256:], fbgb1,
                  fp3[1][0], fp3[1][1])

    Wp1t, pbgb1 = fp2[0]
    l1q = _fp_call(_fp2_body, c1x, c1y, c1z, c2x, c2y, c2z, l2p,
                   [l1, Wp1t[:128], Wp1t[128:], pbgb1,
                    fp2[1][0], fp2[1][1]], 128)

    Wq1t, qbgb1 = fp1[0]
    wxyz1 = Wq1t[:3] + Wq1t[3:6]
    out = _fp_call(_fp1_body, x, y, z, c1x, c1y, c1z, l1q,
                   [wxyz1, Wq1t[6:], qbgb1,
                    fp1[1][0], fp1[1][1],
                    fp1[2][0], fp1[2][1],
                    Wc1.T, cbgb1,
                    Wc2.T, bc2.reshape(1, -1)], 128)
    return out
